# baseline (speedup 1.0000x reference)
import functools
import math

import jax
import jax.numpy as jnp
from jax.experimental import pallas as pl
from jax.experimental.pallas import tpu as pltpu


def _fused_patch_matmul(xa_ref, xb_ref, w_ref, b_ref, o_ref, *, s, wo, rows, sf):
    bias = b_ref[...]
    for q, x_ref in enumerate((xa_ref, xb_ref)):
        acc = None
        for ki in range(s):
            xa = x_ref[:, ki * wo:(ki + 1) * wo, :].reshape(rows, sf)
            d = jnp.dot(xa, w_ref[ki], preferred_element_type=jnp.float32)
            acc = d if acc is None else acc + d
        o_ref[q * rows:(q + 1) * rows, :] = (acc + bias).astype(o_ref.dtype)


def kernel(image_features, images, w_mat, bias2d):
    B, N, F = image_features.shape
    ori_W = int(math.sqrt(N * images.shape[3] // images.shape[2]))
    ori_H = int(ori_W * images.shape[2] // images.shape[3])
    s = 2
    Ho = ori_H // s
    Wo = ori_H // s
    K, O = w_mat.shape
    sf = s * F

    x4 = image_features.reshape(B * Ho, s * Wo, sf)
    w3 = w_mat.reshape(s, sf, O)

    TR = 24
    while (B * Ho) % (2 * TR):
        TR //= 2
    rows = TR * Wo
    grid = ((B * Ho) // (2 * TR),)

    out = pl.pallas_call(
        functools.partial(_fused_patch_matmul, s=s, wo=Wo, rows=rows, sf=sf),
        out_shape=jax.ShapeDtypeStruct((B * Ho * Wo, O), image_features.dtype),
        grid=grid,
        in_specs=[
            pl.BlockSpec((TR, s * Wo, sf), lambda i: (2 * i, 0, 0)),
            pl.BlockSpec((TR, s * Wo, sf), lambda i: (2 * i + 1, 0, 0)),
            pl.BlockSpec((s, sf, O), lambda i: (0, 0, 0)),
            pl.BlockSpec((1, O), lambda i: (0, 0)),
        ],
        out_specs=pl.BlockSpec((2 * rows, O), lambda i: (i, 0)),
        compiler_params=pltpu.CompilerParams(
            dimension_semantics=("parallel",),
            vmem_limit_bytes=60 << 20),
        cost_estimate=pl.CostEstimate(
            flops=2 * B * Ho * Wo * K * O,
            transcendentals=0,
            bytes_accessed=B * N * F * 4 + K * O * 4 + B * Ho * Wo * O * 4),
    )(x4, x4, w3, bias2d)
    return out.reshape(B, Ho * Wo, O)


# speedup vs baseline: 2.2131x; 2.2131x over previous
import functools
import math

import jax
import jax.numpy as jnp
from jax.experimental import pallas as pl
from jax.experimental.pallas import tpu as pltpu


def _fused_patch_matmul(x_ref, w_ref, b_ref, o_ref, *, s, wo, rows, sf):
    swo = s * wo
    acc = None
    for ki in range(s):
        xa = x_ref[:, ki * swo:(ki + 1) * swo, :].reshape(rows, sf)
        d = jnp.dot(xa, w_ref[ki], preferred_element_type=jnp.float32)
        acc = d if acc is None else acc + d
    o_ref[...] = (acc + b_ref[...]).astype(o_ref.dtype)


def kernel(image_features, images, w_mat, bias2d):
    B, N, F = image_features.shape
    ori_W = int(math.sqrt(N * images.shape[3] // images.shape[2]))
    ori_H = int(ori_W * images.shape[2] // images.shape[3])
    s = 2
    Ho = ori_H // s
    Wo = ori_H // s
    K, O = w_mat.shape

    x3 = image_features.reshape(B * Ho, s * s * Wo, F)
    w3 = w_mat.reshape(s, s * F, O)

    TR = 48
    while (B * Ho) % TR:
        TR //= 2
    rows = TR * Wo
    grid = ((B * Ho) // TR,)

    out = pl.pallas_call(
        functools.partial(_fused_patch_matmul, s=s, wo=Wo, rows=rows, sf=s * F),
        out_shape=jax.ShapeDtypeStruct((B * Ho * Wo, O), image_features.dtype),
        grid=grid,
        in_specs=[
            pl.BlockSpec((TR, s * s * Wo, F), lambda i: (i, 0, 0)),
            pl.BlockSpec((s, s * F, O), lambda i: (0, 0, 0)),
            pl.BlockSpec((1, O), lambda i: (0, 0)),
        ],
        out_specs=pl.BlockSpec((rows, O), lambda i: (i, 0)),
        compiler_params=pltpu.CompilerParams(
            dimension_semantics=("parallel",),
            vmem_limit_bytes=60 << 20),
        cost_estimate=pl.CostEstimate(
            flops=2 * B * Ho * Wo * K * O,
            transcendentals=0,
            bytes_accessed=B * N * F * 4 + K * O * 4 + B * Ho * Wo * O * 4),
    )(x3, w3, bias2d)
    return out.reshape(B, Ho * Wo, O)


# speedup vs baseline: 2.2456x; 1.0147x over previous
import functools
import math

import jax
import jax.numpy as jnp
from jax.experimental import pallas as pl
from jax.experimental.pallas import tpu as pltpu


def _fused_patch_matmul(x_ref, w_ref, b_ref, o_ref, *, s, wo, rows, sf):
    swo = s * wo
    acc = None
    for ki in range(s):
        xa = x_ref[:, ki * swo:(ki + 1) * swo, :].reshape(rows, sf)
        d = jnp.dot(xa, w_ref[ki], preferred_element_type=jnp.float32)
        acc = d if acc is None else acc + d
    o_ref[...] = (acc + b_ref[...]).astype(o_ref.dtype)


def kernel(image_features, images, w_mat, bias2d):
    B, N, F = image_features.shape
    ori_W = int(math.sqrt(N * images.shape[3] // images.shape[2]))
    ori_H = int(ori_W * images.shape[2] // images.shape[3])
    s = 2
    Ho = ori_H // s
    Wo = ori_H // s
    K, O = w_mat.shape

    x3 = image_features.reshape(B * Ho, s * s * Wo, F)
    w3 = w_mat.reshape(s, s * F, O)

    TR = 32
    while (B * Ho) % TR:
        TR //= 2
    rows = TR * Wo
    grid = ((B * Ho) // TR,)

    out = pl.pallas_call(
        functools.partial(_fused_patch_matmul, s=s, wo=Wo, rows=rows, sf=s * F),
        out_shape=jax.ShapeDtypeStruct((B * Ho * Wo, O), image_features.dtype),
        grid=grid,
        in_specs=[
            pl.BlockSpec((TR, s * s * Wo, F), lambda i: (i, 0, 0)),
            pl.BlockSpec((s, s * F, O), lambda i: (0, 0, 0)),
            pl.BlockSpec((1, O), lambda i: (0, 0)),
        ],
        out_specs=pl.BlockSpec((rows, O), lambda i: (i, 0)),
        compiler_params=pltpu.CompilerParams(
            dimension_semantics=("parallel",),
            vmem_limit_bytes=60 << 20),
        cost_estimate=pl.CostEstimate(
            flops=2 * B * Ho * Wo * K * O,
            transcendentals=0,
            bytes_accessed=B * N * F * 4 + K * O * 4 + B * Ho * Wo * O * 4),
    )(x3, w3, bias2d)
    return out.reshape(B, Ho * Wo, O)
